# baseline (speedup 1.0000x reference)
import functools

import jax
import jax.numpy as jnp
from jax import lax
from jax.experimental import pallas as pl
from jax.experimental.pallas import tpu as pltpu
from jax.experimental.pallas import tpu_sc as plsc


def _fps_body(npoint, n, xyz_ref, xyzt_ref, out_ref):
    n8 = n // 8
    x = xyz_ref[0, 0]
    y = xyz_ref[0, 1]
    z = xyz_ref[0, 2]
    ids = (lax.broadcasted_iota(jnp.int32, (8, n8), 0) * n8
           + lax.broadcasted_iota(jnp.int32, (8, n8), 1))

    def argmax_first(val, idv):
        rows = 8
        while rows > 1:
            h = rows // 2
            vt, vb = val[:h], val[h:rows]
            it, ib = idv[:h], idv[h:rows]
            cond = (vt > vb) | ((vt == vb) & (it < ib))
            val = jnp.where(cond, vt, vb)
            idv = jnp.where(cond, it, ib)
            rows = h
        w = n8
        while w > 1:
            h = w // 2
            vt, vb = val[:, :h], val[:, h:w]
            it, ib = idv[:, :h], idv[:, h:w]
            cond = (vt > vb) | ((vt == vb) & (it < ib))
            val = jnp.where(cond, vt, vb)
            idv = jnp.where(cond, it, ib)
            w = h
        return idv[0, 0]

    def step(t, carry):
        dists, cur = carry
        row = xyzt_ref[0, pl.ds(cur, 1), :]
        out_ref[0, pl.ds(t, 1), :] = row
        cx = row[0:1, 0:1]
        cy = row[0:1, 1:2]
        cz = row[0:1, 2:3]
        dx = x - cx
        dy = y - cy
        dz = z - cz
        d = dx * dx + dy * dy + dz * dz
        dists = jnp.minimum(dists, d)
        nxt = argmax_first(dists, ids)
        return dists, nxt

    lax.fori_loop(0, npoint, step,
                  (jnp.full((8, n8), 1e10, jnp.float32), jnp.int32(0)))


def _fps(xyz_b3n, xyz_bn3, npoint):
    B, _, n = xyz_b3n.shape
    xr = xyz_b3n.reshape(B, 3, 8, n // 8)
    return pl.pallas_call(
        functools.partial(_fps_body, npoint, n),
        grid=(B,),
        in_specs=[
            pl.BlockSpec((1, 3, 8, n // 8), lambda b: (b, 0, 0, 0)),
            pl.BlockSpec((1, n, 3), lambda b: (b, 0, 0)),
        ],
        out_specs=pl.BlockSpec((1, npoint, 3), lambda b: (b, 0, 0)),
        out_shape=jax.ShapeDtypeStruct((B, npoint, 3), jnp.float32),
    )(xr, xyz_bn3)


def _knn_body(K, n, blkm, q_ref, r_ref, out_ref):
    qx = q_ref[0, :, 0:1]
    qy = q_ref[0, :, 1:2]
    qz = q_ref[0, :, 2:3]
    rx = r_ref[0, 0:1, :]
    ry = r_ref[0, 1:2, :]
    rz = r_ref[0, 2:3, :]
    dx = qx - rx
    dy = qy - ry
    dz = qz - rz
    d = dx * dx + dy * dy + dz * dz
    iota_n = lax.broadcasted_iota(jnp.int32, (blkm, n), 1)
    iota_k = lax.broadcasted_iota(jnp.int32, (blkm, K), 1)
    base = pl.program_id(0) * n
    acc = jnp.zeros((blkm, K), jnp.int32)
    for t in range(K):
        m = jnp.min(d, axis=1, keepdims=True)
        sel = jnp.min(jnp.where(d == m, iota_n, jnp.int32(n)), axis=1,
                      keepdims=True)
        acc = jnp.where(iota_k == t, sel + base, acc)
        d = jnp.where(iota_n == sel, jnp.inf, d)
    out_ref[0] = acc


def _knn(q_bm3, r_b3n, K, blkm):
    B, M, _ = q_bm3.shape
    n = r_b3n.shape[2]
    return pl.pallas_call(
        functools.partial(_knn_body, K, n, blkm),
        grid=(B, M // blkm),
        in_specs=[
            pl.BlockSpec((1, blkm, 3), lambda b, mb: (b, mb, 0)),
            pl.BlockSpec((1, 3, n), lambda b, mb: (b, 0, 0)),
        ],
        out_specs=pl.BlockSpec((1, blkm, K), lambda b, mb: (b, mb, 0)),
        out_shape=jax.ShapeDtypeStruct((B, M, K), jnp.int32),
    )(q_bm3, r_b3n)


def _sc_gather(table, idx):
    G = idx.shape[0]
    Dp = table.shape[1]
    NW = 32
    bpw = G // NW
    ch = min(bpw, 128)
    nch = bpw // ch
    idx2 = idx.reshape(G // ch, ch)
    mesh = plsc.VectorSubcoreMesh(core_axis_name="c", subcore_axis_name="s")

    @functools.partial(
        pl.kernel,
        out_type=jax.ShapeDtypeStruct((G, Dp), jnp.float32),
        mesh=mesh,
        scratch_types=[
            pltpu.VMEM((nch, ch), jnp.int32),
            pltpu.VMEM((ch, Dp), jnp.float32),
            pltpu.SemaphoreType.DMA,
        ],
    )
    def gk(table_hbm, idx_hbm, out_hbm, idx_v, rows_v, sem):
        wid = lax.axis_index("s") * 2 + lax.axis_index("c")
        pltpu.sync_copy(idx_hbm.at[pl.ds(wid * nch, nch)], idx_v)
        for j in range(nch):
            pltpu.async_copy(table_hbm.at[idx_v.at[j]], rows_v, sem).wait()
            pltpu.sync_copy(
                rows_v, out_hbm.at[pl.ds(wid * bpw + j * ch, ch)])

    return gk(table, idx2)


def _mk_table(feat_bnc, xyz_bn3, dp):
    B, N, C = feat_bnc.shape
    t = jnp.concatenate([feat_bnc, xyz_bn3], axis=-1)
    t = jnp.pad(t, ((0, 0), (0, 0), (0, dp - C - 3)))
    return t.reshape(B * N, dp)


def _qpad(q, C, Dp, blkm):
    cols = lax.broadcasted_iota(jnp.int32, (blkm, Dp), 1)
    return (jnp.where(cols == C, q[:, 0:1], 0.0)
            + jnp.where(cols == C + 1, q[:, 1:2], 0.0)
            + jnp.where(cols == C + 2, q[:, 2:3], 0.0))


def _setconv_body(K, C, blkm, q_ref, g_ref, w1_ref, b1_ref, w2_ref, b2_ref,
                  out_ref):
    qp = _qpad(q_ref[0], C, g_ref.shape[3], blkm)
    acc = None
    for k in range(K):
        h1 = jnp.dot(g_ref[0, k] - qp, w1_ref[...],
                     preferred_element_type=jnp.float32) + b1_ref[...]
        h1 = jnp.maximum(h1, 0.0)
        h2 = jnp.dot(h1, w2_ref[...],
                     preferred_element_type=jnp.float32) + b2_ref[...]
        h2 = jnp.maximum(h2, 0.0)
        acc = h2 if acc is None else jnp.maximum(acc, h2)
    out_ref[0] = acc


def _setconv_mlp(g, q, layers, C, blkm):
    B, K, M, Dp = g.shape
    (W1, b1), (W2, b2) = layers
    H1, H2 = W1.shape[1], W2.shape[1]
    w1p = jnp.zeros((Dp, H1), jnp.float32).at[:C + 3].set(W1)
    return pl.pallas_call(
        functools.partial(_setconv_body, K, C, blkm),
        grid=(B, M // blkm),
        in_specs=[
            pl.BlockSpec((1, blkm, 3), lambda b, mb: (b, mb, 0)),
            pl.BlockSpec((1, K, blkm, Dp), lambda b, mb: (b, 0, mb, 0)),
            pl.BlockSpec((Dp, H1), lambda b, mb: (0, 0)),
            pl.BlockSpec((1, H1), lambda b, mb: (0, 0)),
            pl.BlockSpec((H1, H2), lambda b, mb: (0, 0)),
            pl.BlockSpec((1, H2), lambda b, mb: (0, 0)),
        ],
        out_specs=pl.BlockSpec((1, blkm, H2), lambda b, mb: (b, mb, 0)),
        out_shape=jax.ShapeDtypeStruct((B, M, H2), jnp.float32),
    )(q, g, w1p, b1[None], W2, b2[None])


def _upconv_body(K, C, blkm, q_ref, g_ref, fd_ref, w1_ref, b1_ref, w2a_ref,
                 w2b_ref, b2_ref, out_ref):
    qp = _qpad(q_ref[0], C, g_ref.shape[3], blkm)
    acc = None
    for k in range(K):
        h1 = jnp.dot(g_ref[0, k] - qp, w1_ref[...],
                     preferred_element_type=jnp.float32) + b1_ref[...]
        h1 = jnp.maximum(h1, 0.0)
        acc = h1 if acc is None else jnp.maximum(acc, h1)
    h2 = (jnp.dot(acc, w2a_ref[...], preferred_element_type=jnp.float32)
          + jnp.dot(fd_ref[0], w2b_ref[...],
                    preferred_element_type=jnp.float32)
          + b2_ref[...])
    out_ref[0] = jnp.maximum(h2, 0.0)


def _upconv_mlp(g, q, fd, m1, m2, C, blkm):
    B, K, M, Dp = g.shape
    Cdn = fd.shape[2]
    (W1, b1), = m1
    (W2, b2), = m2
    H1 = W1.shape[1]
    H2 = W2.shape[1]
    w1p = jnp.zeros((Dp, H1), jnp.float32).at[:C + 3].set(W1)
    w2a = W2[:H1]
    w2b = W2[H1:]
    return pl.pallas_call(
        functools.partial(_upconv_body, K, C, blkm),
        grid=(B, M // blkm),
        in_specs=[
            pl.BlockSpec((1, blkm, 3), lambda b, mb: (b, mb, 0)),
            pl.BlockSpec((1, K, blkm, Dp), lambda b, mb: (b, 0, mb, 0)),
            pl.BlockSpec((1, blkm, Cdn), lambda b, mb: (b, mb, 0)),
            pl.BlockSpec((Dp, H1), lambda b, mb: (0, 0)),
            pl.BlockSpec((1, H1), lambda b, mb: (0, 0)),
            pl.BlockSpec((H1, H2), lambda b, mb: (0, 0)),
            pl.BlockSpec((Cdn, H2), lambda b, mb: (0, 0)),
            pl.BlockSpec((1, H2), lambda b, mb: (0, 0)),
        ],
        out_specs=pl.BlockSpec((1, blkm, H2), lambda b, mb: (b, mb, 0)),
        out_shape=jax.ShapeDtypeStruct((B, M, H2), jnp.float32),
    )(q, g, fd, w1p, b1[None], w2a, w2b, b2[None])


def _dense0_body(f_ref, w_ref, b_ref, out_ref):
    out_ref[0] = jnp.maximum(
        jnp.dot(f_ref[0], w_ref[...], preferred_element_type=jnp.float32)
        + b_ref[...], 0.0)


def _dense0(f_bn3, W, b, blk):
    B, N, _ = f_bn3.shape
    H = W.shape[1]
    return pl.pallas_call(
        _dense0_body,
        grid=(B, N // blk),
        in_specs=[
            pl.BlockSpec((1, blk, 3), lambda b_, nb: (b_, nb, 0)),
            pl.BlockSpec((3, H), lambda b_, nb: (0, 0)),
            pl.BlockSpec((1, H), lambda b_, nb: (0, 0)),
        ],
        out_specs=pl.BlockSpec((1, blk, H), lambda b_, nb: (b_, nb, 0)),
        out_shape=jax.ShapeDtypeStruct((B, N, H), jnp.float32),
    )(f_bn3, W, b[None])


def _set_conv_level(xyz_b3n, xyz_bn3, feat_bnc, npoint, K, layers, dp,
                    knn_blk, mlp_blk):
    B, _, N = xyz_b3n.shape
    C = feat_bnc.shape[2]
    nxyz_bm3 = _fps(xyz_b3n, xyz_bn3, npoint)
    nxyz_b3m = jnp.transpose(nxyz_bm3, (0, 2, 1))
    nidx = _knn(nxyz_bm3, xyz_b3n, K, knn_blk)
    table = _mk_table(feat_bnc, xyz_bn3, dp)
    gidx = jnp.transpose(nidx, (0, 2, 1)).reshape(-1)
    g = _sc_gather(table, gidx).reshape(B, K, npoint, dp)
    nf = _setconv_mlp(g, nxyz_bm3, layers, C, mlp_blk)
    return nxyz_b3m, nxyz_bm3, nf


def _set_up_conv_level(xyz_sp_b3n, xyz_sp_bn3, xyz_dn_bm3, feat_sp, feat_dn,
                       K, m1, m2, dp, knn_blk, mlp_blk):
    B = xyz_sp_b3n.shape[0]
    M = xyz_dn_bm3.shape[1]
    C = feat_sp.shape[2]
    nidx = _knn(xyz_dn_bm3, xyz_sp_b3n, K, knn_blk)
    table = _mk_table(feat_sp, xyz_sp_bn3, dp)
    gidx = jnp.transpose(nidx, (0, 2, 1)).reshape(-1)
    g = _sc_gather(table, gidx).reshape(B, K, M, dp)
    return _upconv_mlp(g, xyz_dn_bm3, feat_dn, m1, m2, C, mlp_blk)


def kernel(pc, feat, params):
    B = pc.shape[0]
    xyz_b3n = pc
    xyz_bn3 = jnp.transpose(pc, (0, 2, 1))
    W0, b0 = params['conv0']
    f0 = _dense0(jnp.transpose(feat, (0, 2, 1)), W0, b0, 1024)

    x1_b3m, x1_bm3, f1 = _set_conv_level(
        xyz_b3n, xyz_bn3, f0, 2048, 16, params['d1'], 128, 256, 256)
    x2_b3m, x2_bm3, f2 = _set_conv_level(
        x1_b3m, x1_bm3, f1, 512, 16, params['d2'], 128, 512, 256)
    x3_b3m, x3_bm3, f3 = _set_conv_level(
        x2_b3m, x2_bm3, f2, 128, 16, params['d3'], 256, 128, 128)
    x4_b3m, x4_bm3, f4 = _set_conv_level(
        x3_b3m, x3_bm3, f3, 64, 16, params['d4'], 256, 64, 64)

    u3 = _set_up_conv_level(x4_b3m, x4_bm3, x3_bm3, f4, f3, 8,
                            params['u4m1'], params['u4m2'], 256, 128, 128)
    u2 = _set_up_conv_level(x3_b3m, x3_bm3, x2_bm3, u3, f2, 8,
                            params['u3m1'], params['u3m2'], 256, 512, 256)
    u1 = _set_up_conv_level(x2_b3m, x2_bm3, x1_bm3, u2, f1, 8,
                            params['u2m1'], params['u2m2'], 256, 256, 256)
    u0 = _set_up_conv_level(x1_b3m, x1_bm3, xyz_bn3, u1, f0, 8,
                            params['u1m1'], params['u1m2'], 128, 256, 512)

    return (x1_bm3, x2_bm3, x3_bm3, u0, u1, u2, u3)


# speedup vs baseline: 1.3287x; 1.3287x over previous
import functools

import jax
import jax.numpy as jnp
from jax import lax
from jax.experimental import pallas as pl
from jax.experimental.pallas import tpu as pltpu
from jax.experimental.pallas import tpu_sc as plsc


def _fps_body(npoint, n, xyz_ref, xyzt_ref, out_ref):
    n8 = n // 8
    x = xyz_ref[0, 0]
    y = xyz_ref[0, 1]
    z = xyz_ref[0, 2]
    ids = (lax.broadcasted_iota(jnp.int32, (8, n8), 0) * n8
           + lax.broadcasted_iota(jnp.int32, (8, n8), 1))

    def step(t, carry):
        dists, cur = carry
        row = xyzt_ref[0, pl.ds(cur, 1), :]
        out_ref[0, pl.ds(t, 1), :] = row
        cx = row[0:1, 0:1]
        cy = row[0:1, 1:2]
        cz = row[0:1, 2:3]
        dx = x - cx
        dy = y - cy
        dz = z - cz
        d = dx * dx + dy * dy + dz * dz
        dists = jnp.minimum(dists, d)
        m = jnp.max(dists)
        nxt = jnp.min(jnp.where(dists == m, ids, jnp.int32(n)))
        return dists, nxt

    lax.fori_loop(0, npoint, step,
                  (jnp.full((8, n8), 1e10, jnp.float32), jnp.int32(0)))


def _fps(xyz_b3n, xyz_bn3, npoint):
    B, _, n = xyz_b3n.shape
    xr = xyz_b3n.reshape(B, 3, 8, n // 8)
    return pl.pallas_call(
        functools.partial(_fps_body, npoint, n),
        grid=(B,),
        in_specs=[
            pl.BlockSpec((1, 3, 8, n // 8), lambda b: (b, 0, 0, 0)),
            pl.BlockSpec((1, n, 3), lambda b: (b, 0, 0)),
        ],
        out_specs=pl.BlockSpec((1, npoint, 3), lambda b: (b, 0, 0)),
        out_shape=jax.ShapeDtypeStruct((B, npoint, 3), jnp.float32),
    )(xr, xyz_bn3)


def _knn_body(K, n, blkm, q_ref, r_ref, out_ref):
    qx = q_ref[0, :, 0:1]
    qy = q_ref[0, :, 1:2]
    qz = q_ref[0, :, 2:3]
    rx = r_ref[0, 0:1, :]
    ry = r_ref[0, 1:2, :]
    rz = r_ref[0, 2:3, :]
    dx = qx - rx
    dy = qy - ry
    dz = qz - rz
    d = dx * dx + dy * dy + dz * dz
    iota_n = lax.broadcasted_iota(jnp.int32, (blkm, n), 1)
    iota_k = lax.broadcasted_iota(jnp.int32, (blkm, K), 1)
    base = pl.program_id(0) * n
    acc = jnp.zeros((blkm, K), jnp.int32)
    for t in range(K):
        m = jnp.min(d, axis=1, keepdims=True)
        sel = jnp.min(jnp.where(d == m, iota_n, jnp.int32(n)), axis=1,
                      keepdims=True)
        acc = jnp.where(iota_k == t, sel + base, acc)
        d = jnp.where(iota_n == sel, jnp.inf, d)
    out_ref[0] = acc


def _knn(q_bm3, r_b3n, K, blkm):
    B, M, _ = q_bm3.shape
    n = r_b3n.shape[2]
    return pl.pallas_call(
        functools.partial(_knn_body, K, n, blkm),
        grid=(B, M // blkm),
        in_specs=[
            pl.BlockSpec((1, blkm, 3), lambda b, mb: (b, mb, 0)),
            pl.BlockSpec((1, 3, n), lambda b, mb: (b, 0, 0)),
        ],
        out_specs=pl.BlockSpec((1, blkm, K), lambda b, mb: (b, mb, 0)),
        out_shape=jax.ShapeDtypeStruct((B, M, K), jnp.int32),
    )(q_bm3, r_b3n)


def _sc_gather(table, idx):
    G = idx.shape[0]
    Dp = table.shape[1]
    NW = 32
    bpw = G // NW
    ch = min(bpw, 128)
    nch = bpw // ch
    NB = min(3 if Dp > 128 else 4, nch)
    idx2 = idx.reshape(G // ch, ch)
    mesh = plsc.VectorSubcoreMesh(core_axis_name="c", subcore_axis_name="s")

    scratch = [pltpu.VMEM((nch, ch), jnp.int32)]
    scratch += [pltpu.VMEM((ch, Dp), jnp.float32) for _ in range(NB)]
    scratch += [pltpu.SemaphoreType.DMA for _ in range(2 * NB)]

    @functools.partial(
        pl.kernel,
        out_type=jax.ShapeDtypeStruct((G, Dp), jnp.float32),
        mesh=mesh,
        scratch_types=scratch,
    )
    def gk(table_hbm, idx_hbm, out_hbm, *scr):
        idx_v = scr[0]
        rows = scr[1:1 + NB]
        gsem = scr[1 + NB:1 + 2 * NB]
        osem = scr[1 + 2 * NB:1 + 3 * NB]
        wid = lax.axis_index("s") * 2 + lax.axis_index("c")
        pltpu.sync_copy(idx_hbm.at[pl.ds(wid * nch, nch)], idx_v)
        gh = {}
        oh = {}

        def start_gather(j):
            b = j % NB
            gh[j] = pltpu.async_copy(
                table_hbm.at[idx_v.at[j]], rows[b], gsem[b])

        start_gather(0)
        for j in range(nch):
            b = j % NB
            if j + 1 < nch:
                if j + 1 >= NB:
                    oh[j + 1 - NB].wait()
                start_gather(j + 1)
            gh[j].wait()
            oh[j] = pltpu.async_copy(
                rows[b], out_hbm.at[pl.ds(wid * bpw + j * ch, ch)], osem[b])
        for j in range(max(0, nch - NB), nch):
            oh[j].wait()

    return gk(table, idx2)


def _mk_table(feat_bnc, xyz_bn3, dp):
    B, N, C = feat_bnc.shape
    t = jnp.concatenate([feat_bnc, xyz_bn3], axis=-1)
    t = jnp.pad(t, ((0, 0), (0, 0), (0, dp - C - 3)))
    return t.reshape(B * N, dp)


def _qpad(q, C, Dp, blkm):
    cols = lax.broadcasted_iota(jnp.int32, (blkm, Dp), 1)
    return (jnp.where(cols == C, q[:, 0:1], 0.0)
            + jnp.where(cols == C + 1, q[:, 1:2], 0.0)
            + jnp.where(cols == C + 2, q[:, 2:3], 0.0))


def _setconv_body(K, C, blkm, q_ref, g_ref, w1_ref, b1_ref, w2_ref, b2_ref,
                  out_ref):
    qp = _qpad(q_ref[0], C, g_ref.shape[3], blkm)
    acc = None
    for k in range(K):
        h1 = jnp.dot(g_ref[0, :, k, :] - qp, w1_ref[...],
                     preferred_element_type=jnp.float32) + b1_ref[...]
        h1 = jnp.maximum(h1, 0.0)
        h2 = jnp.dot(h1, w2_ref[...],
                     preferred_element_type=jnp.float32) + b2_ref[...]
        h2 = jnp.maximum(h2, 0.0)
        acc = h2 if acc is None else jnp.maximum(acc, h2)
    out_ref[0] = acc


def _setconv_mlp(g, q, layers, C, blkm):
    B, M, K, Dp = g.shape
    (W1, b1), (W2, b2) = layers
    H1, H2 = W1.shape[1], W2.shape[1]
    w1p = jnp.zeros((Dp, H1), jnp.float32).at[:C + 3].set(W1)
    return pl.pallas_call(
        functools.partial(_setconv_body, K, C, blkm),
        grid=(B, M // blkm),
        in_specs=[
            pl.BlockSpec((1, blkm, 3), lambda b, mb: (b, mb, 0)),
            pl.BlockSpec((1, blkm, K, Dp), lambda b, mb: (b, mb, 0, 0)),
            pl.BlockSpec((Dp, H1), lambda b, mb: (0, 0)),
            pl.BlockSpec((1, H1), lambda b, mb: (0, 0)),
            pl.BlockSpec((H1, H2), lambda b, mb: (0, 0)),
            pl.BlockSpec((1, H2), lambda b, mb: (0, 0)),
        ],
        out_specs=pl.BlockSpec((1, blkm, H2), lambda b, mb: (b, mb, 0)),
        out_shape=jax.ShapeDtypeStruct((B, M, H2), jnp.float32),
    )(q, g, w1p, b1[None], W2, b2[None])


def _upconv_body(K, C, blkm, q_ref, g_ref, fd_ref, w1_ref, b1_ref, w2a_ref,
                 w2b_ref, b2_ref, out_ref):
    qp = _qpad(q_ref[0], C, g_ref.shape[3], blkm)
    acc = None
    for k in range(K):
        h1 = jnp.dot(g_ref[0, :, k, :] - qp, w1_ref[...],
                     preferred_element_type=jnp.float32) + b1_ref[...]
        h1 = jnp.maximum(h1, 0.0)
        acc = h1 if acc is None else jnp.maximum(acc, h1)
    h2 = (jnp.dot(acc, w2a_ref[...], preferred_element_type=jnp.float32)
          + jnp.dot(fd_ref[0], w2b_ref[...],
                    preferred_element_type=jnp.float32)
          + b2_ref[...])
    out_ref[0] = jnp.maximum(h2, 0.0)


def _upconv_mlp(g, q, fd, m1, m2, C, blkm):
    B, M, K, Dp = g.shape
    Cdn = fd.shape[2]
    (W1, b1), = m1
    (W2, b2), = m2
    H1 = W1.shape[1]
    H2 = W2.shape[1]
    w1p = jnp.zeros((Dp, H1), jnp.float32).at[:C + 3].set(W1)
    w2a = W2[:H1]
    w2b = W2[H1:]
    return pl.pallas_call(
        functools.partial(_upconv_body, K, C, blkm),
        grid=(B, M // blkm),
        in_specs=[
            pl.BlockSpec((1, blkm, 3), lambda b, mb: (b, mb, 0)),
            pl.BlockSpec((1, blkm, K, Dp), lambda b, mb: (b, mb, 0, 0)),
            pl.BlockSpec((1, blkm, Cdn), lambda b, mb: (b, mb, 0)),
            pl.BlockSpec((Dp, H1), lambda b, mb: (0, 0)),
            pl.BlockSpec((1, H1), lambda b, mb: (0, 0)),
            pl.BlockSpec((H1, H2), lambda b, mb: (0, 0)),
            pl.BlockSpec((Cdn, H2), lambda b, mb: (0, 0)),
            pl.BlockSpec((1, H2), lambda b, mb: (0, 0)),
        ],
        out_specs=pl.BlockSpec((1, blkm, H2), lambda b, mb: (b, mb, 0)),
        out_shape=jax.ShapeDtypeStruct((B, M, H2), jnp.float32),
    )(q, g, fd, w1p, b1[None], w2a, w2b, b2[None])


def _dense0_body(f_ref, w_ref, b_ref, out_ref):
    out_ref[0] = jnp.maximum(
        jnp.dot(f_ref[0], w_ref[...], preferred_element_type=jnp.float32)
        + b_ref[...], 0.0)


def _dense0(f_bn3, W, b, blk):
    B, N, _ = f_bn3.shape
    H = W.shape[1]
    return pl.pallas_call(
        _dense0_body,
        grid=(B, N // blk),
        in_specs=[
            pl.BlockSpec((1, blk, 3), lambda b_, nb: (b_, nb, 0)),
            pl.BlockSpec((3, H), lambda b_, nb: (0, 0)),
            pl.BlockSpec((1, H), lambda b_, nb: (0, 0)),
        ],
        out_specs=pl.BlockSpec((1, blk, H), lambda b_, nb: (b_, nb, 0)),
        out_shape=jax.ShapeDtypeStruct((B, N, H), jnp.float32),
    )(f_bn3, W, b[None])


def _set_conv_level(xyz_b3n, xyz_bn3, feat_bnc, npoint, K, layers, dp,
                    knn_blk, mlp_blk):
    B, _, N = xyz_b3n.shape
    C = feat_bnc.shape[2]
    nxyz_bm3 = _fps(xyz_b3n, xyz_bn3, npoint)
    nxyz_b3m = jnp.transpose(nxyz_bm3, (0, 2, 1))
    nidx = _knn(nxyz_bm3, xyz_b3n, K, knn_blk)
    table = _mk_table(feat_bnc, xyz_bn3, dp)
    g = _sc_gather(table, nidx.reshape(-1)).reshape(B, npoint, K, dp)
    nf = _setconv_mlp(g, nxyz_bm3, layers, C, mlp_blk)
    return nxyz_b3m, nxyz_bm3, nf


def _set_up_conv_level(xyz_sp_b3n, xyz_sp_bn3, xyz_dn_bm3, feat_sp, feat_dn,
                       K, m1, m2, dp, knn_blk, mlp_blk):
    B = xyz_sp_b3n.shape[0]
    M = xyz_dn_bm3.shape[1]
    C = feat_sp.shape[2]
    nidx = _knn(xyz_dn_bm3, xyz_sp_b3n, K, knn_blk)
    table = _mk_table(feat_sp, xyz_sp_bn3, dp)
    g = _sc_gather(table, nidx.reshape(-1)).reshape(B, M, K, dp)
    return _upconv_mlp(g, xyz_dn_bm3, feat_dn, m1, m2, C, mlp_blk)


def kernel(pc, feat, params):
    B = pc.shape[0]
    xyz_b3n = pc
    xyz_bn3 = jnp.transpose(pc, (0, 2, 1))
    W0, b0 = params['conv0']
    f0 = _dense0(jnp.transpose(feat, (0, 2, 1)), W0, b0, 1024)

    x1_b3m, x1_bm3, f1 = _set_conv_level(
        xyz_b3n, xyz_bn3, f0, 2048, 16, params['d1'], 128, 256, 256)
    x2_b3m, x2_bm3, f2 = _set_conv_level(
        x1_b3m, x1_bm3, f1, 512, 16, params['d2'], 128, 512, 256)
    x3_b3m, x3_bm3, f3 = _set_conv_level(
        x2_b3m, x2_bm3, f2, 128, 16, params['d3'], 256, 128, 128)
    x4_b3m, x4_bm3, f4 = _set_conv_level(
        x3_b3m, x3_bm3, f3, 64, 16, params['d4'], 256, 64, 64)

    u3 = _set_up_conv_level(x4_b3m, x4_bm3, x3_bm3, f4, f3, 8,
                            params['u4m1'], params['u4m2'], 256, 128, 128)
    u2 = _set_up_conv_level(x3_b3m, x3_bm3, x2_bm3, u3, f2, 8,
                            params['u3m1'], params['u3m2'], 256, 512, 256)
    u1 = _set_up_conv_level(x2_b3m, x2_bm3, x1_bm3, u2, f1, 8,
                            params['u2m1'], params['u2m2'], 256, 256, 256)
    u0 = _set_up_conv_level(x1_b3m, x1_bm3, xyz_bn3, u1, f0, 8,
                            params['u1m1'], params['u1m2'], 128, 256, 512)

    return (x1_bm3, x2_bm3, x3_bm3, u0, u1, u2, u3)


# speedup vs baseline: 1.4339x; 1.0792x over previous
import functools

import jax
import jax.numpy as jnp
from jax import lax
from jax.experimental import pallas as pl
from jax.experimental.pallas import tpu as pltpu
from jax.experimental.pallas import tpu_sc as plsc


def _fps_body(npoint, n, B, xyz_ref, xyzt_ref, out_ref):
    n8 = n // 8
    xyzs = [(xyz_ref[b, 0], xyz_ref[b, 1], xyz_ref[b, 2]) for b in range(B)]
    ids = (lax.broadcasted_iota(jnp.int32, (8, n8), 0) * n8
           + lax.broadcasted_iota(jnp.int32, (8, n8), 1))

    def step(t, carry):
        out = []
        for b in range(B):
            dists, cur = carry[b]
            x, y, z = xyzs[b]
            row = xyzt_ref[b, pl.ds(cur, 1), :]
            out_ref[b, pl.ds(t, 1), :] = row
            cx = row[0:1, 0:1]
            cy = row[0:1, 1:2]
            cz = row[0:1, 2:3]
            dx = x - cx
            dy = y - cy
            dz = z - cz
            d = dx * dx + dy * dy + dz * dz
            dists = jnp.minimum(dists, d)
            m = jnp.max(dists)
            nxt = jnp.min(jnp.where(dists == m, ids, jnp.int32(n)))
            out.append((dists, nxt))
        return tuple(out)

    init = tuple((jnp.full((8, n8), 1e10, jnp.float32), jnp.int32(0))
                 for _ in range(B))
    lax.fori_loop(0, npoint, step, init)


def _fps(xyz_b3n, xyz_bn3, npoint):
    B, _, n = xyz_b3n.shape
    xr = xyz_b3n.reshape(B, 3, 8, n // 8)
    return pl.pallas_call(
        functools.partial(_fps_body, npoint, n, B),
        out_shape=jax.ShapeDtypeStruct((B, npoint, 3), jnp.float32),
    )(xr, xyz_bn3)


def _knn_body(K, n, blkm, q_ref, r_ref, out_ref):
    qx = q_ref[0, :, 0:1]
    qy = q_ref[0, :, 1:2]
    qz = q_ref[0, :, 2:3]
    rx = r_ref[0, 0:1, :]
    ry = r_ref[0, 1:2, :]
    rz = r_ref[0, 2:3, :]
    dx = qx - rx
    dy = qy - ry
    dz = qz - rz
    d = dx * dx + dy * dy + dz * dz
    iota_n = lax.broadcasted_iota(jnp.int32, (blkm, n), 1)
    iota_k = lax.broadcasted_iota(jnp.int32, (blkm, K), 1)
    base = pl.program_id(0) * n
    acc = jnp.zeros((blkm, K), jnp.int32)
    for t in range(K):
        m = jnp.min(d, axis=1, keepdims=True)
        sel = jnp.min(jnp.where(d == m, iota_n, jnp.int32(n)), axis=1,
                      keepdims=True)
        acc = jnp.where(iota_k == t, sel + base, acc)
        d = jnp.where(iota_n == sel, jnp.inf, d)
    out_ref[0] = acc


def _knn(q_bm3, r_b3n, K, blkm):
    B, M, _ = q_bm3.shape
    n = r_b3n.shape[2]
    return pl.pallas_call(
        functools.partial(_knn_body, K, n, blkm),
        grid=(B, M // blkm),
        in_specs=[
            pl.BlockSpec((1, blkm, 3), lambda b, mb: (b, mb, 0)),
            pl.BlockSpec((1, 3, n), lambda b, mb: (b, 0, 0)),
        ],
        out_specs=pl.BlockSpec((1, blkm, K), lambda b, mb: (b, mb, 0)),
        out_shape=jax.ShapeDtypeStruct((B, M, K), jnp.int32),
    )(q_bm3, r_b3n)


def _sc_gather(table, idx):
    G = idx.shape[0]
    Dp = table.shape[1]
    NW = 32
    bpw = G // NW
    ch = min(bpw, 128)
    nch = bpw // ch
    NB = min(3 if Dp > 128 else 4, nch)
    idx2 = idx.reshape(G // ch, ch)
    mesh = plsc.VectorSubcoreMesh(core_axis_name="c", subcore_axis_name="s")

    scratch = [pltpu.VMEM((nch, ch), jnp.int32)]
    scratch += [pltpu.VMEM((ch, Dp), jnp.float32) for _ in range(NB)]
    scratch += [pltpu.SemaphoreType.DMA for _ in range(2 * NB)]

    @functools.partial(
        pl.kernel,
        out_type=jax.ShapeDtypeStruct((G, Dp), jnp.float32),
        mesh=mesh,
        scratch_types=scratch,
    )
    def gk(table_hbm, idx_hbm, out_hbm, *scr):
        idx_v = scr[0]
        rows = scr[1:1 + NB]
        gsem = scr[1 + NB:1 + 2 * NB]
        osem = scr[1 + 2 * NB:1 + 3 * NB]
        wid = lax.axis_index("s") * 2 + lax.axis_index("c")
        pltpu.sync_copy(idx_hbm.at[pl.ds(wid * nch, nch)], idx_v)
        gh = {}
        oh = {}

        def start_gather(j):
            b = j % NB
            gh[j] = pltpu.async_copy(
                table_hbm.at[idx_v.at[j]], rows[b], gsem[b])

        start_gather(0)
        for j in range(nch):
            b = j % NB
            if j + 1 < nch:
                if j + 1 >= NB:
                    oh[j + 1 - NB].wait()
                start_gather(j + 1)
            gh[j].wait()
            oh[j] = pltpu.async_copy(
                rows[b], out_hbm.at[pl.ds(wid * bpw + j * ch, ch)], osem[b])
        for j in range(max(0, nch - NB), nch):
            oh[j].wait()

    return gk(table, idx2)


def _mk_table(feat_bnc, xyz_bn3, dp):
    B, N, C = feat_bnc.shape
    t = jnp.concatenate([feat_bnc, xyz_bn3], axis=-1)
    t = jnp.pad(t, ((0, 0), (0, 0), (0, dp - C - 3)))
    return t.reshape(B * N, dp)


def _qpad(q, C, Dp, blkm):
    cols = lax.broadcasted_iota(jnp.int32, (blkm, Dp), 1)
    return (jnp.where(cols == C, q[:, 0:1], 0.0)
            + jnp.where(cols == C + 1, q[:, 1:2], 0.0)
            + jnp.where(cols == C + 2, q[:, 2:3], 0.0))


def _setconv_body(K, C, blkm, q_ref, g_ref, w1_ref, b1_ref, w2_ref, b2_ref,
                  out_ref):
    qp = _qpad(q_ref[0], C, g_ref.shape[3], blkm)
    acc = None
    for k in range(K):
        h1 = jnp.dot(g_ref[0, :, k, :] - qp, w1_ref[...],
                     preferred_element_type=jnp.float32) + b1_ref[...]
        h1 = jnp.maximum(h1, 0.0)
        h2 = jnp.dot(h1, w2_ref[...],
                     preferred_element_type=jnp.float32) + b2_ref[...]
        h2 = jnp.maximum(h2, 0.0)
        acc = h2 if acc is None else jnp.maximum(acc, h2)
    out_ref[0] = acc


def _setconv_mlp(g, q, layers, C, blkm):
    B, M, K, Dp = g.shape
    (W1, b1), (W2, b2) = layers
    H1, H2 = W1.shape[1], W2.shape[1]
    w1p = jnp.zeros((Dp, H1), jnp.float32).at[:C + 3].set(W1)
    return pl.pallas_call(
        functools.partial(_setconv_body, K, C, blkm),
        grid=(B, M // blkm),
        in_specs=[
            pl.BlockSpec((1, blkm, 3), lambda b, mb: (b, mb, 0)),
            pl.BlockSpec((1, blkm, K, Dp), lambda b, mb: (b, mb, 0, 0)),
            pl.BlockSpec((Dp, H1), lambda b, mb: (0, 0)),
            pl.BlockSpec((1, H1), lambda b, mb: (0, 0)),
            pl.BlockSpec((H1, H2), lambda b, mb: (0, 0)),
            pl.BlockSpec((1, H2), lambda b, mb: (0, 0)),
        ],
        out_specs=pl.BlockSpec((1, blkm, H2), lambda b, mb: (b, mb, 0)),
        out_shape=jax.ShapeDtypeStruct((B, M, H2), jnp.float32),
    )(q, g, w1p, b1[None], W2, b2[None])


def _upconv_body(K, C, blkm, q_ref, g_ref, fd_ref, w1_ref, b1_ref, w2a_ref,
                 w2b_ref, b2_ref, out_ref):
    qp = _qpad(q_ref[0], C, g_ref.shape[3], blkm)
    acc = None
    for k in range(K):
        h1 = jnp.dot(g_ref[0, :, k, :] - qp, w1_ref[...],
                     preferred_element_type=jnp.float32) + b1_ref[...]
        h1 = jnp.maximum(h1, 0.0)
        acc = h1 if acc is None else jnp.maximum(acc, h1)
    h2 = (jnp.dot(acc, w2a_ref[...], preferred_element_type=jnp.float32)
          + jnp.dot(fd_ref[0], w2b_ref[...],
                    preferred_element_type=jnp.float32)
          + b2_ref[...])
    out_ref[0] = jnp.maximum(h2, 0.0)


def _upconv_mlp(g, q, fd, m1, m2, C, blkm):
    B, M, K, Dp = g.shape
    Cdn = fd.shape[2]
    (W1, b1), = m1
    (W2, b2), = m2
    H1 = W1.shape[1]
    H2 = W2.shape[1]
    w1p = jnp.zeros((Dp, H1), jnp.float32).at[:C + 3].set(W1)
    w2a = W2[:H1]
    w2b = W2[H1:]
    return pl.pallas_call(
        functools.partial(_upconv_body, K, C, blkm),
        grid=(B, M // blkm),
        in_specs=[
            pl.BlockSpec((1, blkm, 3), lambda b, mb: (b, mb, 0)),
            pl.BlockSpec((1, blkm, K, Dp), lambda b, mb: (b, mb, 0, 0)),
            pl.BlockSpec((1, blkm, Cdn), lambda b, mb: (b, mb, 0)),
            pl.BlockSpec((Dp, H1), lambda b, mb: (0, 0)),
            pl.BlockSpec((1, H1), lambda b, mb: (0, 0)),
            pl.BlockSpec((H1, H2), lambda b, mb: (0, 0)),
            pl.BlockSpec((Cdn, H2), lambda b, mb: (0, 0)),
            pl.BlockSpec((1, H2), lambda b, mb: (0, 0)),
        ],
        out_specs=pl.BlockSpec((1, blkm, H2), lambda b, mb: (b, mb, 0)),
        out_shape=jax.ShapeDtypeStruct((B, M, H2), jnp.float32),
    )(q, g, fd, w1p, b1[None], w2a, w2b, b2[None])


def _dense0_body(f_ref, w_ref, b_ref, out_ref):
    out_ref[0] = jnp.maximum(
        jnp.dot(f_ref[0], w_ref[...], preferred_element_type=jnp.float32)
        + b_ref[...], 0.0)


def _dense0(f_bn3, W, b, blk):
    B, N, _ = f_bn3.shape
    H = W.shape[1]
    return pl.pallas_call(
        _dense0_body,
        grid=(B, N // blk),
        in_specs=[
            pl.BlockSpec((1, blk, 3), lambda b_, nb: (b_, nb, 0)),
            pl.BlockSpec((3, H), lambda b_, nb: (0, 0)),
            pl.BlockSpec((1, H), lambda b_, nb: (0, 0)),
        ],
        out_specs=pl.BlockSpec((1, blk, H), lambda b_, nb: (b_, nb, 0)),
        out_shape=jax.ShapeDtypeStruct((B, N, H), jnp.float32),
    )(f_bn3, W, b[None])


def _set_conv_level(xyz_b3n, xyz_bn3, feat_bnc, npoint, K, layers, dp,
                    knn_blk, mlp_blk):
    B, _, N = xyz_b3n.shape
    C = feat_bnc.shape[2]
    nxyz_bm3 = _fps(xyz_b3n, xyz_bn3, npoint)
    nxyz_b3m = jnp.transpose(nxyz_bm3, (0, 2, 1))
    nidx = _knn(nxyz_bm3, xyz_b3n, K, knn_blk)
    table = _mk_table(feat_bnc, xyz_bn3, dp)
    g = _sc_gather(table, nidx.reshape(-1)).reshape(B, npoint, K, dp)
    nf = _setconv_mlp(g, nxyz_bm3, layers, C, mlp_blk)
    return nxyz_b3m, nxyz_bm3, nf


def _set_up_conv_level(xyz_sp_b3n, xyz_sp_bn3, xyz_dn_bm3, feat_sp, feat_dn,
                       K, m1, m2, dp, knn_blk, mlp_blk):
    B = xyz_sp_b3n.shape[0]
    M = xyz_dn_bm3.shape[1]
    C = feat_sp.shape[2]
    nidx = _knn(xyz_dn_bm3, xyz_sp_b3n, K, knn_blk)
    table = _mk_table(feat_sp, xyz_sp_bn3, dp)
    g = _sc_gather(table, nidx.reshape(-1)).reshape(B, M, K, dp)
    return _upconv_mlp(g, xyz_dn_bm3, feat_dn, m1, m2, C, mlp_blk)


def kernel(pc, feat, params):
    B = pc.shape[0]
    xyz_b3n = pc
    xyz_bn3 = jnp.transpose(pc, (0, 2, 1))
    W0, b0 = params['conv0']
    f0 = _dense0(jnp.transpose(feat, (0, 2, 1)), W0, b0, 1024)

    x1_b3m, x1_bm3, f1 = _set_conv_level(
        xyz_b3n, xyz_bn3, f0, 2048, 16, params['d1'], 128, 256, 256)
    x2_b3m, x2_bm3, f2 = _set_conv_level(
        x1_b3m, x1_bm3, f1, 512, 16, params['d2'], 128, 512, 256)
    x3_b3m, x3_bm3, f3 = _set_conv_level(
        x2_b3m, x2_bm3, f2, 128, 16, params['d3'], 256, 128, 128)
    x4_b3m, x4_bm3, f4 = _set_conv_level(
        x3_b3m, x3_bm3, f3, 64, 16, params['d4'], 256, 64, 64)

    u3 = _set_up_conv_level(x4_b3m, x4_bm3, x3_bm3, f4, f3, 8,
                            params['u4m1'], params['u4m2'], 256, 128, 128)
    u2 = _set_up_conv_level(x3_b3m, x3_bm3, x2_bm3, u3, f2, 8,
                            params['u3m1'], params['u3m2'], 256, 512, 256)
    u1 = _set_up_conv_level(x2_b3m, x2_bm3, x1_bm3, u2, f1, 8,
                            params['u2m1'], params['u2m2'], 256, 256, 256)
    u0 = _set_up_conv_level(x1_b3m, x1_bm3, xyz_bn3, u1, f0, 8,
                            params['u1m1'], params['u1m2'], 128, 256, 512)

    return (x1_bm3, x2_bm3, x3_bm3, u0, u1, u2, u3)
